# baseline (speedup 1.0000x reference)
import jax
import jax.numpy as jnp
from jax.experimental import pallas as pl


def kernel(rel_logit, obj_logit, rel_pair_idx, boxes):
    obj_class_prob = jax.nn.softmax(obj_logit, axis=-1)
    obj_class_prob = obj_class_prob.at[:, 0].set(0.0)
    obj_scores = jnp.max(obj_class_prob[:, 1:], axis=1)
    obj_pred = jnp.argmax(obj_class_prob[:, 1:], axis=1) + 1
    obj_scores0 = obj_scores[rel_pair_idx[:, 0]]
    obj_scores1 = obj_scores[rel_pair_idx[:, 1]]
    rel_class_prob = jax.nn.softmax(rel_logit, axis=-1)
    rel_max_scores = jnp.max(rel_class_prob[:, 1:], axis=1)
    rel_class = jnp.argmax(rel_class_prob[:, 1:], axis=1) + 1
    triple_scores = rel_max_scores * obj_scores0 * obj_scores1
    sorting_idx = jnp.argsort(-triple_scores)
    rel_pair_sorted = jnp.take(rel_pair_idx, sorting_idx, axis=0)
    rel_class_prob_sorted = jnp.take(rel_class_prob, sorting_idx, axis=0)
    rel_max_scores_sorted = jnp.take(rel_max_scores, sorting_idx, axis=0)
    rel_labels_sorted = jnp.take(rel_class, sorting_idx, axis=0)
    return (boxes, obj_pred, obj_scores, rel_pair_sorted, rel_class_prob_sorted,
            rel_labels_sorted, rel_max_scores_sorted)


# speedup vs baseline: 1.3230x; 1.3230x over previous
import functools

import jax
import jax.numpy as jnp
from jax import lax
from jax.experimental import pallas as pl
from jax.experimental.pallas import tpu as pltpu
from jax.experimental.pallas import tpu_sc as plsc

NREL = 20000
NPAD = 20480
RB = 1024
NBLK = NPAD // RB
KEY_ONE = 0x3F800000

NT = 16
CH = NPAD // NT
CL = CH // 16
NPASS = 6


def _row_sum_chunk8(e, n):
    width = ((n + 7) // 8) * 8
    epad = jnp.pad(e, ((0, 0), (0, width - n)))
    acc = epad[:, 0:8]
    for j in range(1, width // 8):
        acc = acc + epad[:, 8 * j:8 * j + 8]
    a4 = acc[:, 0:4] + acc[:, 4:8]
    a2 = a4[:, 0:2] + a4[:, 2:4]
    return a2[:, 0:1] + a2[:, 1:2]


def _obj_kernel(obj_logit_ref, scores_ref, pred_ref):
    x = obj_logit_ref[...]
    m = jnp.max(x, axis=1, keepdims=True)
    e = jnp.exp(x - m)
    s = _row_sum_chunk8(e, 151)
    prob = e / s
    lane = lax.broadcasted_iota(jnp.int32, prob.shape, 1)
    fg = jnp.where(lane >= 1, prob, -1.0)
    mx = jnp.max(fg, axis=1, keepdims=True)
    cand = jnp.where(fg == mx, lane, 10_000)
    pred = jnp.min(cand, axis=1, keepdims=True)
    scores_ref[...] = mx
    pred_ref[...] = pred


def _rel_kernel(rel_logit_ref, pair_ref, objrow_ref,
                stag_ref, keys_ref):
    p = pl.program_id(0)
    x = rel_logit_ref[...]
    m = jnp.max(x, axis=1, keepdims=True)
    e = jnp.exp(x - m)
    s = _row_sum_chunk8(e, 51)
    prob = e / s
    lane = lax.broadcasted_iota(jnp.int32, prob.shape, 1)
    fg = jnp.where(lane >= 1, prob, -1.0)
    rmax = jnp.max(fg, axis=1, keepdims=True)
    cand = jnp.where(fg == rmax, lane, 10_000)
    label = jnp.min(cand, axis=1, keepdims=True)

    table = objrow_ref[0:1, :]
    tlane = lax.broadcasted_iota(jnp.int32, (1, 1024), 1)
    idx0 = pair_ref[:, 0:1]
    idx1 = pair_ref[:, 1:2]
    s0 = jnp.sum(jnp.where(idx0 == tlane, table, 0.0), axis=1, keepdims=True)
    s1 = jnp.sum(jnp.where(idx1 == tlane, table, 0.0), axis=1, keepdims=True)
    t = (rmax * s0) * s1

    row_id = p * RB + lax.broadcasted_iota(jnp.int32, (RB, 1), 0)
    valid = row_id < NREL
    key = jnp.where(valid, KEY_ONE - lax.bitcast_convert_type(t, jnp.int32),
                    KEY_ONE)
    keys_ref[...] = key

    olane = lax.broadcasted_iota(jnp.int32, (RB, 128), 1)
    prob_pad = jnp.pad(prob, ((0, 0), (0, 77)))
    bc0 = lax.bitcast_convert_type(idx0, jnp.float32)
    bc1 = lax.bitcast_convert_type(idx1, jnp.float32)
    bcl = lax.bitcast_convert_type(label, jnp.float32)
    out = jnp.where(olane < 51, prob_pad, jnp.broadcast_to(rmax, (RB, 128)))
    out = jnp.where(olane == 52, jnp.broadcast_to(bc0, (RB, 128)), out)
    out = jnp.where(olane == 53, jnp.broadcast_to(bc1, (RB, 128)), out)
    out = jnp.where(olane == 54, jnp.broadcast_to(bcl, (RB, 128)), out)
    stag_ref[...] = out


def _tc_prep(rel_logit_p, pair_p, obj_logit):
    obj_scores2d, obj_pred2d = pl.pallas_call(
        _obj_kernel,
        out_shape=(
            jax.ShapeDtypeStruct((1000, 1), jnp.float32),
            jax.ShapeDtypeStruct((1000, 1), jnp.int32),
        ),
    )(obj_logit)

    obj_row = jnp.zeros((8, 1024), jnp.float32).at[0, :1000].set(
        obj_scores2d[:, 0])

    stag, keys2d = pl.pallas_call(
        _rel_kernel,
        grid=(NBLK,),
        in_specs=[
            pl.BlockSpec((RB, 51), lambda i: (i, 0)),
            pl.BlockSpec((RB, 2), lambda i: (i, 0)),
            pl.BlockSpec((8, 1024), lambda i: (0, 0)),
        ],
        out_specs=(
            pl.BlockSpec((RB, 128), lambda i: (i, 0)),
            pl.BlockSpec((RB, 1), lambda i: (i, 0)),
        ),
        out_shape=(
            jax.ShapeDtypeStruct((NPAD, 128), jnp.float32),
            jax.ShapeDtypeStruct((NPAD, 1), jnp.int32),
        ),
    )(rel_logit_p, pair_p, obj_row)
    return obj_scores2d, obj_pred2d, stag, keys2d


def _sc_sort_body(keys_hbm, iota_hbm, stag_hbm,
                  out_hbm,
                  kc, vc, dc, hist, offs, totT_v, own_tot,
                  rows, half_idx,
                  kA, vA, kB, vB, totT_sp, sem):
    tid = lax.axis_index("s")
    base = tid * CH
    lane = lax.broadcasted_iota(jnp.int32, (16,), 0)

    if True:
        for p in range(NPASS):
            shift = 5 * p
            k_in, v_in = (kA, vA) if p % 2 == 0 else (kB, vB)
            k_out, v_out = (kB, vB) if p % 2 == 0 else (kA, vA)
            if p == 0:
                pltpu.sync_copy(keys_hbm.at[pl.ds(base, CH)], kc)
                pltpu.sync_copy(iota_hbm.at[pl.ds(base, CH)], vc)
            else:
                pltpu.sync_copy(k_in.at[pl.ds(base, CH)], kc)
                pltpu.sync_copy(v_in.at[pl.ds(base, CH)], vc)

            for j in range(32):
                hist[pl.ds(j * 16, 16)] = jnp.zeros((16,), jnp.int32)

            def hbody(t, _, shift=shift):
                idx = lane * CL + t
                k = plsc.load_gather(kc, [idx])
                d = (k >> shift) & 31
                hidx = d * 16 + lane
                cur = plsc.load_gather(hist, [hidx])
                plsc.store_scatter(hist, [hidx], cur + 1)
                return 0
            lax.fori_loop(0, CL, hbody, 0)

            for h in range(2):
                acc = jnp.zeros((16,), jnp.int32)
                for l in range(16):
                    acc = acc + plsc.load_gather(
                        hist, [(lane + h * 16) * 16 + l])
                own_tot[pl.ds(h * 16, 16)] = acc
            pltpu.sync_copy(own_tot, totT_sp.at[pl.ds(tid * 32, 32)])
            plsc.subcore_barrier()
            pltpu.sync_copy(totT_sp, totT_v)

            base_s = jnp.int32(0)
            for d in range(32):
                trow = plsc.load_gather(totT_v, [lane * 32 + d])
                myp = jnp.sum(jnp.where(lane < tid, trow, 0))
                tot_d = jnp.sum(trow)
                own = hist[pl.ds(d * 16, 16)]
                excl = plsc.cumsum(own) - own
                offs[pl.ds(d * 16, 16)] = excl + (base_s + myp)
                base_s = base_s + tot_d

            def pbody(t, _, shift=shift):
                idx = lane * CL + t
                k = plsc.load_gather(kc, [idx])
                d = (k >> shift) & 31
                hidx = d * 16 + lane
                off = plsc.load_gather(offs, [hidx])
                plsc.store_scatter(offs, [hidx], off + 1)
                plsc.store_scatter(dc, [idx], off)
                return 0
            lax.fori_loop(0, CL, pbody, 0)

            pltpu.sync_copy(kc, k_out.at[dc])
            pltpu.sync_copy(vc, v_out.at[dc])
            plsc.subcore_barrier()

        for h in range(2):
            hb = base + h * (CH // 2)
            pltpu.sync_copy(vA.at[pl.ds(hb, CH // 2)], half_idx)
            pltpu.async_copy(stag_hbm.at[half_idx], rows, sem).wait()
            pltpu.sync_copy(rows, out_hbm.at[pl.ds(hb, CH // 2)])


def _sc_sort(keys, iota, stag):
    mesh = plsc.VectorSubcoreMesh(core_axis_name="c", subcore_axis_name="s",
                                  num_cores=1)
    f = pl.kernel(
        _sc_sort_body,
        out_type=jax.ShapeDtypeStruct((NPAD, 128), jnp.float32),
        mesh=mesh,
        compiler_params=pltpu.CompilerParams(needs_layout_passes=False),
        scratch_types=[
            pltpu.VMEM((CH,), jnp.int32),
            pltpu.VMEM((CH,), jnp.int32),
            pltpu.VMEM((CH,), jnp.int32),
            pltpu.VMEM((512,), jnp.int32),
            pltpu.VMEM((512,), jnp.int32),
            pltpu.VMEM((512,), jnp.int32),
            pltpu.VMEM((32,), jnp.int32),
            pltpu.VMEM((CH // 2, 128), jnp.float32),
            pltpu.VMEM((CH // 2,), jnp.int32),
            pltpu.VMEM_SHARED((NPAD,), jnp.int32),
            pltpu.VMEM_SHARED((NPAD,), jnp.int32),
            pltpu.VMEM_SHARED((NPAD,), jnp.int32),
            pltpu.VMEM_SHARED((NPAD,), jnp.int32),
            pltpu.VMEM_SHARED((512,), jnp.int32),
            pltpu.SemaphoreType.DMA,
        ],
    )
    return f(keys, iota, stag)


def kernel(rel_logit, obj_logit, rel_pair_idx, boxes):
    rel_logit_p = jnp.pad(rel_logit, ((0, NPAD - NREL), (0, 0)))
    pair_p = jnp.pad(rel_pair_idx, ((0, NPAD - NREL), (0, 0)))

    obj_scores2d, obj_pred2d, stag, keys2d = _tc_prep(
        rel_logit_p, pair_p, obj_logit)

    keys = keys2d[:, 0]
    iota = jnp.arange(NPAD, dtype=jnp.int32)
    sorted_rows = _sc_sort(keys, iota, stag)

    rel_class_prob_sorted = sorted_rows[:NREL, :51]
    rel_max_scores_sorted = sorted_rows[:NREL, 51]
    rel_pair_sorted = lax.bitcast_convert_type(
        sorted_rows[:NREL, 52:54], jnp.int32)
    rel_labels_sorted = lax.bitcast_convert_type(
        sorted_rows[:NREL, 54], jnp.int32)
    return (boxes, obj_pred2d[:, 0], obj_scores2d[:, 0], rel_pair_sorted,
            rel_class_prob_sorted, rel_labels_sorted, rel_max_scores_sorted)


# speedup vs baseline: 1.4654x; 1.1076x over previous
import functools

import jax
import jax.numpy as jnp
from jax import lax
from jax.experimental import pallas as pl
from jax.experimental.pallas import tpu as pltpu
from jax.experimental.pallas import tpu_sc as plsc

NREL = 20000
NPAD = 20480
RB = 1000
NBLK = NREL // RB
KEY_ONE = 0x3F800000

NT = 16
CH = NPAD // NT
CL = CH // 16
NPASS = 6


def _row_sum_chunk8(e, n):
    width = ((n + 7) // 8) * 8
    epad = jnp.pad(e, ((0, 0), (0, width - n)))
    acc = epad[:, 0:8]
    for j in range(1, width // 8):
        acc = acc + epad[:, 8 * j:8 * j + 8]
    a4 = acc[:, 0:4] + acc[:, 4:8]
    a2 = a4[:, 0:2] + a4[:, 2:4]
    return a2[:, 0:1] + a2[:, 1:2]


def _obj_kernel(obj_logit_ref, scores_ref, pred_ref):
    x = obj_logit_ref[...]
    m = jnp.max(x, axis=1, keepdims=True)
    e = jnp.exp(x - m)
    s = _row_sum_chunk8(e, 151)
    prob = e / s
    lane = lax.broadcasted_iota(jnp.int32, prob.shape, 1)
    fg = jnp.where(lane >= 1, prob, -1.0)
    mx = jnp.max(fg, axis=1, keepdims=True)
    cand = jnp.where(fg == mx, lane, 10_000)
    pred = jnp.min(cand, axis=1, keepdims=True)
    scores_ref[...] = mx
    pred_ref[...] = pred


def _rel_kernel(rel_logit_ref, pair_ref, stag_ref, rmax_ref):
    x = rel_logit_ref[...]
    m = jnp.max(x, axis=1, keepdims=True)
    e = jnp.exp(x - m)
    s = _row_sum_chunk8(e, 51)
    prob = e / s
    lane = lax.broadcasted_iota(jnp.int32, prob.shape, 1)
    fg = jnp.where(lane >= 1, prob, -1.0)
    rmax = jnp.max(fg, axis=1, keepdims=True)
    cand = jnp.where(fg == rmax, lane, 10_000)
    label = jnp.min(cand, axis=1, keepdims=True)
    rmax_ref[...] = rmax

    idx0 = pair_ref[:, 0:1]
    idx1 = pair_ref[:, 1:2]
    olane = lax.broadcasted_iota(jnp.int32, (RB, 128), 1)
    prob_pad = jnp.pad(prob, ((0, 0), (0, 77)))
    bc0 = lax.bitcast_convert_type(idx0, jnp.float32)
    bc1 = lax.bitcast_convert_type(idx1, jnp.float32)
    bcl = lax.bitcast_convert_type(label, jnp.float32)
    out = jnp.where(olane < 51, prob_pad, jnp.broadcast_to(rmax, (RB, 128)))
    out = jnp.where(olane == 52, jnp.broadcast_to(bc0, (RB, 128)), out)
    out = jnp.where(olane == 53, jnp.broadcast_to(bc1, (RB, 128)), out)
    out = jnp.where(olane == 54, jnp.broadcast_to(bcl, (RB, 128)), out)
    stag_ref[...] = out


def _tc_prep(rel_logit_p, pair_p, obj_logit):
    obj_scores2d, obj_pred2d = pl.pallas_call(
        _obj_kernel,
        out_shape=(
            jax.ShapeDtypeStruct((1000, 1), jnp.float32),
            jax.ShapeDtypeStruct((1000, 1), jnp.int32),
        ),
    )(obj_logit)

    stag, rmax2d = pl.pallas_call(
        _rel_kernel,
        grid=(NBLK,),
        in_specs=[
            pl.BlockSpec((RB, 51), lambda i: (i, 0)),
            pl.BlockSpec((RB, 2), lambda i: (i, 0)),
        ],
        out_specs=(
            pl.BlockSpec((RB, 128), lambda i: (i, 0)),
            pl.BlockSpec((RB, 1), lambda i: (i, 0)),
        ),
        out_shape=(
            jax.ShapeDtypeStruct((NPAD, 128), jnp.float32),
            jax.ShapeDtypeStruct((NREL, 1), jnp.float32),
        ),
    )(rel_logit_p, pair_p)
    return obj_scores2d, obj_pred2d, stag, rmax2d


def _sc_sort_body(table_hbm, rmax_hbm, pair0_hbm, pair1_hbm, stag_hbm,
                  out_hbm,
                  kc, vc, dc, hist, offs, totT_v, own_tot,
                  p0_v, p1_v, rm_v, table_v,
                  rows, half_idx,
                  kA, vA, kB, vB, totT_sp, sem):
    tid = lax.axis_index("s")
    base = tid * CH
    lane = lax.broadcasted_iota(jnp.int32, (16,), 0)

    if True:
        pltpu.sync_copy(table_hbm, table_v)
        pltpu.sync_copy(rmax_hbm.at[pl.ds(base, CH)], rm_v)
        pltpu.sync_copy(pair0_hbm.at[pl.ds(base, CH)], p0_v)
        pltpu.sync_copy(pair1_hbm.at[pl.ds(base, CH)], p1_v)

        def kbody(t, _):
            idx = lane * CL + t
            i0 = plsc.load_gather(p0_v, [idx])
            i1 = plsc.load_gather(p1_v, [idx])
            s0 = plsc.load_gather(table_v, [i0])
            s1 = plsc.load_gather(table_v, [i1])
            rm = plsc.load_gather(rm_v, [idx])
            tv = (rm * s0) * s1
            key = KEY_ONE - plsc.bitcast(tv, jnp.int32)
            plsc.store_scatter(kc, [idx], key)
            plsc.store_scatter(vc, [idx], base + idx)
            return 0
        lax.fori_loop(0, CL, kbody, 0)

        for p in range(NPASS):
            shift = 5 * p
            k_in, v_in = (kA, vA) if p % 2 == 0 else (kB, vB)
            k_out, v_out = (kB, vB) if p % 2 == 0 else (kA, vA)
            if p == 0:
                pass
            else:
                pltpu.sync_copy(k_in.at[pl.ds(base, CH)], kc)
                pltpu.sync_copy(v_in.at[pl.ds(base, CH)], vc)

            for j in range(32):
                hist[pl.ds(j * 16, 16)] = jnp.zeros((16,), jnp.int32)

            def hbody(t, _, shift=shift):
                idx = lane * CL + t
                k = plsc.load_gather(kc, [idx])
                d = (k >> shift) & 31
                hidx = d * 16 + lane
                cur = plsc.load_gather(hist, [hidx])
                plsc.store_scatter(hist, [hidx], cur + 1)
                return 0
            lax.fori_loop(0, CL, hbody, 0)

            for h in range(2):
                acc = jnp.zeros((16,), jnp.int32)
                for l in range(16):
                    acc = acc + plsc.load_gather(
                        hist, [(lane + h * 16) * 16 + l])
                own_tot[pl.ds(h * 16, 16)] = acc
            pltpu.sync_copy(own_tot, totT_sp.at[pl.ds(tid * 32, 32)])
            plsc.subcore_barrier()
            pltpu.sync_copy(totT_sp, totT_v)

            base_s = jnp.int32(0)
            for d in range(32):
                trow = plsc.load_gather(totT_v, [lane * 32 + d])
                myp = jnp.sum(jnp.where(lane < tid, trow, 0))
                tot_d = jnp.sum(trow)
                own = hist[pl.ds(d * 16, 16)]
                excl = plsc.cumsum(own) - own
                offs[pl.ds(d * 16, 16)] = excl + (base_s + myp)
                base_s = base_s + tot_d

            def pbody(t, _, shift=shift):
                idx = lane * CL + t
                k = plsc.load_gather(kc, [idx])
                d = (k >> shift) & 31
                hidx = d * 16 + lane
                off = plsc.load_gather(offs, [hidx])
                plsc.store_scatter(offs, [hidx], off + 1)
                plsc.store_scatter(dc, [idx], off)
                return 0
            lax.fori_loop(0, CL, pbody, 0)

            pltpu.sync_copy(kc, k_out.at[dc])
            pltpu.sync_copy(vc, v_out.at[dc])
            plsc.subcore_barrier()

        for h in range(2):
            hb = base + h * (CH // 2)
            pltpu.sync_copy(vA.at[pl.ds(hb, CH // 2)], half_idx)
            pltpu.async_copy(stag_hbm.at[half_idx], rows, sem).wait()
            pltpu.sync_copy(rows, out_hbm.at[pl.ds(hb, CH // 2)])


def _sc_sort(table, rmax_p, pair0_p, pair1_p, stag):
    mesh = plsc.VectorSubcoreMesh(core_axis_name="c", subcore_axis_name="s",
                                  num_cores=1)
    f = pl.kernel(
        _sc_sort_body,
        out_type=jax.ShapeDtypeStruct((NPAD, 128), jnp.float32),
        mesh=mesh,
        compiler_params=pltpu.CompilerParams(needs_layout_passes=False),
        scratch_types=[
            pltpu.VMEM((CH,), jnp.int32),
            pltpu.VMEM((CH,), jnp.int32),
            pltpu.VMEM((CH,), jnp.int32),
            pltpu.VMEM((512,), jnp.int32),
            pltpu.VMEM((512,), jnp.int32),
            pltpu.VMEM((512,), jnp.int32),
            pltpu.VMEM((32,), jnp.int32),
            pltpu.VMEM((CH,), jnp.int32),
            pltpu.VMEM((CH,), jnp.int32),
            pltpu.VMEM((CH,), jnp.float32),
            pltpu.VMEM((1024,), jnp.float32),
            pltpu.VMEM((CH // 2, 128), jnp.float32),
            pltpu.VMEM((CH // 2,), jnp.int32),
            pltpu.VMEM_SHARED((NPAD,), jnp.int32),
            pltpu.VMEM_SHARED((NPAD,), jnp.int32),
            pltpu.VMEM_SHARED((NPAD,), jnp.int32),
            pltpu.VMEM_SHARED((NPAD,), jnp.int32),
            pltpu.VMEM_SHARED((512,), jnp.int32),
            pltpu.SemaphoreType.DMA,
        ],
    )
    return f(table, rmax_p, pair0_p, pair1_p, stag)


def kernel(rel_logit, obj_logit, rel_pair_idx, boxes):
    obj_scores2d, obj_pred2d, stag, rmax2d = _tc_prep(
        rel_logit, rel_pair_idx, obj_logit)

    table = jnp.pad(obj_scores2d[:, 0], (0, 24))
    rmax_p = jnp.pad(rmax2d[:, 0], (0, NPAD - NREL))
    pair0_p = jnp.pad(rel_pair_idx[:, 0], (0, NPAD - NREL))
    pair1_p = jnp.pad(rel_pair_idx[:, 1], (0, NPAD - NREL))
    sorted_rows = _sc_sort(table, rmax_p, pair0_p, pair1_p, stag)

    rel_class_prob_sorted = sorted_rows[:NREL, :51]
    rel_max_scores_sorted = sorted_rows[:NREL, 51]
    rel_pair_sorted = lax.bitcast_convert_type(
        sorted_rows[:NREL, 52:54], jnp.int32)
    rel_labels_sorted = lax.bitcast_convert_type(
        sorted_rows[:NREL, 54], jnp.int32)
    return (boxes, obj_pred2d[:, 0], obj_scores2d[:, 0], rel_pair_sorted,
            rel_class_prob_sorted, rel_labels_sorted, rel_max_scores_sorted)


# speedup vs baseline: 1.4827x; 1.0118x over previous
import functools

import jax
import jax.numpy as jnp
from jax import lax
from jax.experimental import pallas as pl
from jax.experimental.pallas import tpu as pltpu
from jax.experimental.pallas import tpu_sc as plsc

NREL = 20000
NPAD = 20480
RB = 1000
NBLK = NREL // RB
KEY_ONE = 0x3F800000

NT = 16
CH = NPAD // NT
CL = CH // 16
NPASS = 6


def _row_sum_chunk8(e, n):
    width = ((n + 7) // 8) * 8
    epad = jnp.pad(e, ((0, 0), (0, width - n)))
    acc = epad[:, 0:8]
    for j in range(1, width // 8):
        acc = acc + epad[:, 8 * j:8 * j + 8]
    a4 = acc[:, 0:4] + acc[:, 4:8]
    a2 = a4[:, 0:2] + a4[:, 2:4]
    return a2[:, 0:1] + a2[:, 1:2]


def _obj_kernel(obj_logit_ref, scores_ref, pred_ref):
    x = obj_logit_ref[...]
    m = jnp.max(x, axis=1, keepdims=True)
    e = jnp.exp(x - m)
    s = _row_sum_chunk8(e, 151)
    prob = e / s
    lane = lax.broadcasted_iota(jnp.int32, prob.shape, 1)
    fg = jnp.where(lane >= 1, prob, -1.0)
    mx = jnp.max(fg, axis=1, keepdims=True)
    cand = jnp.where(fg == mx, lane, 10_000)
    pred = jnp.min(cand, axis=1, keepdims=True)
    scores_ref[...] = mx
    pred_ref[...] = pred


def _rel_kernel(rel_logit_ref, pair_ref, stag_ref, rmax_ref):
    x = rel_logit_ref[...]
    m = jnp.max(x, axis=1, keepdims=True)
    e = jnp.exp(x - m)
    s = _row_sum_chunk8(e, 51)
    prob = e / s
    lane = lax.broadcasted_iota(jnp.int32, prob.shape, 1)
    fg = jnp.where(lane >= 1, prob, -1.0)
    rmax = jnp.max(fg, axis=1, keepdims=True)
    cand = jnp.where(fg == rmax, lane, 10_000)
    label = jnp.min(cand, axis=1, keepdims=True)
    rmax_ref[...] = rmax

    idx0 = pair_ref[:, 0:1]
    idx1 = pair_ref[:, 1:2]
    olane = lax.broadcasted_iota(jnp.int32, (RB, 128), 1)
    prob_pad = jnp.pad(prob, ((0, 0), (0, 77)))
    bc0 = lax.bitcast_convert_type(idx0, jnp.float32)
    bc1 = lax.bitcast_convert_type(idx1, jnp.float32)
    bcl = lax.bitcast_convert_type(label, jnp.float32)
    out = jnp.where(olane < 51, prob_pad, jnp.broadcast_to(rmax, (RB, 128)))
    out = jnp.where(olane == 52, jnp.broadcast_to(bc0, (RB, 128)), out)
    out = jnp.where(olane == 53, jnp.broadcast_to(bc1, (RB, 128)), out)
    out = jnp.where(olane == 54, jnp.broadcast_to(bcl, (RB, 128)), out)
    stag_ref[...] = out


def _tc_prep(rel_logit_p, pair_p, obj_logit):
    obj_scores2d, obj_pred2d = pl.pallas_call(
        _obj_kernel,
        out_shape=(
            jax.ShapeDtypeStruct((1000, 1), jnp.float32),
            jax.ShapeDtypeStruct((1000, 1), jnp.int32),
        ),
    )(obj_logit)

    stag, rmax2d = pl.pallas_call(
        _rel_kernel,
        grid=(NBLK,),
        in_specs=[
            pl.BlockSpec((RB, 51), lambda i: (i, 0)),
            pl.BlockSpec((RB, 2), lambda i: (i, 0)),
        ],
        out_specs=(
            pl.BlockSpec((RB, 128), lambda i: (i, 0)),
            pl.BlockSpec((RB, 1), lambda i: (i, 0)),
        ),
        out_shape=(
            jax.ShapeDtypeStruct((NPAD, 128), jnp.float32),
            jax.ShapeDtypeStruct((NREL, 1), jnp.float32),
        ),
    )(rel_logit_p, pair_p)
    return obj_scores2d, obj_pred2d, stag, rmax2d


def _sc_sort_body(table_hbm, rmax_hbm, pair0_hbm, pair1_hbm, stag_hbm,
                  out_hbm,
                  kc, vc, dc, hist, offs, totT_v, own_tot,
                  p0_v, p1_v, rm_v, table_v,
                  rows, rows2, half_idx, half_idx2,
                  kA, vA, kB, vB, totT_sp, sem, sem2):
    tid = lax.axis_index("s")
    base = tid * CH
    lane = lax.broadcasted_iota(jnp.int32, (16,), 0)

    if True:
        pltpu.sync_copy(table_hbm, table_v)
        pltpu.sync_copy(rmax_hbm.at[pl.ds(base, CH)], rm_v)
        pltpu.sync_copy(pair0_hbm.at[pl.ds(base, CH)], p0_v)
        pltpu.sync_copy(pair1_hbm.at[pl.ds(base, CH)], p1_v)

        def kbody(t, _):
            idx = lane * CL + t
            i0 = plsc.load_gather(p0_v, [idx])
            i1 = plsc.load_gather(p1_v, [idx])
            s0 = plsc.load_gather(table_v, [i0])
            s1 = plsc.load_gather(table_v, [i1])
            rm = plsc.load_gather(rm_v, [idx])
            tv = (rm * s0) * s1
            key = KEY_ONE - plsc.bitcast(tv, jnp.int32)
            plsc.store_scatter(kc, [idx], key)
            plsc.store_scatter(vc, [idx], base + idx)
            return 0
        lax.fori_loop(0, CL, kbody, 0)

        for p in range(NPASS):
            shift = 5 * p
            k_in, v_in = (kA, vA) if p % 2 == 0 else (kB, vB)
            k_out, v_out = (kB, vB) if p % 2 == 0 else (kA, vA)
            if p == 0:
                pass
            else:
                pltpu.sync_copy(k_in.at[pl.ds(base, CH)], kc)
                pltpu.sync_copy(v_in.at[pl.ds(base, CH)], vc)

            for j in range(32):
                hist[pl.ds(j * 16, 16)] = jnp.zeros((16,), jnp.int32)

            def hbody(t, _, shift=shift):
                idx = lane * CL + t
                k = plsc.load_gather(kc, [idx])
                d = (k >> shift) & 31
                hidx = d * 16 + lane
                cur = plsc.load_gather(hist, [hidx])
                plsc.store_scatter(hist, [hidx], cur + 1)
                return 0
            lax.fori_loop(0, CL, hbody, 0)

            for h in range(2):
                acc = jnp.zeros((16,), jnp.int32)
                for l in range(16):
                    acc = acc + plsc.load_gather(
                        hist, [(lane + h * 16) * 16 + l])
                own_tot[pl.ds(h * 16, 16)] = acc
            pltpu.sync_copy(own_tot, totT_sp.at[pl.ds(tid * 32, 32)])
            plsc.subcore_barrier()
            pltpu.sync_copy(totT_sp, totT_v)

            base_s = jnp.int32(0)
            for d in range(32):
                trow = plsc.load_gather(totT_v, [lane * 32 + d])
                myp = jnp.sum(jnp.where(lane < tid, trow, 0))
                tot_d = jnp.sum(trow)
                own = hist[pl.ds(d * 16, 16)]
                excl = plsc.cumsum(own) - own
                offs[pl.ds(d * 16, 16)] = excl + (base_s + myp)
                base_s = base_s + tot_d

            def pbody(t, _, shift=shift):
                idx = lane * CL + t
                k = plsc.load_gather(kc, [idx])
                d = (k >> shift) & 31
                hidx = d * 16 + lane
                off = plsc.load_gather(offs, [hidx])
                plsc.store_scatter(offs, [hidx], off + 1)
                plsc.store_scatter(dc, [idx], off)
                return 0
            lax.fori_loop(0, CL, pbody, 0)

            if p < NPASS - 1:
                pltpu.sync_copy(kc, k_out.at[dc])
            pltpu.sync_copy(vc, v_out.at[dc])
            plsc.subcore_barrier()

        Q = CH // 4
        bufs = (rows, rows2)
        idxs = (half_idx, half_idx2)
        sems = (sem, sem2)

        def start(q):
            b = q % 2
            pltpu.sync_copy(vA.at[pl.ds(base + q * Q, Q)], idxs[b])
            return pltpu.async_copy(stag_hbm.at[idxs[b]], bufs[b], sems[b])

        d_prev = start(0)
        for q in range(1, 4):
            d_next = start(q)
            d_prev.wait()
            pltpu.sync_copy(bufs[(q - 1) % 2],
                            out_hbm.at[pl.ds(base + (q - 1) * Q, Q)])
            d_prev = d_next
        d_prev.wait()
        pltpu.sync_copy(bufs[1], out_hbm.at[pl.ds(base + 3 * Q, Q)])


def _sc_sort(table, rmax_p, pair0_p, pair1_p, stag):
    mesh = plsc.VectorSubcoreMesh(core_axis_name="c", subcore_axis_name="s",
                                  num_cores=1)
    f = pl.kernel(
        _sc_sort_body,
        out_type=jax.ShapeDtypeStruct((NPAD, 128), jnp.float32),
        mesh=mesh,
        compiler_params=pltpu.CompilerParams(needs_layout_passes=False),
        scratch_types=[
            pltpu.VMEM((CH,), jnp.int32),
            pltpu.VMEM((CH,), jnp.int32),
            pltpu.VMEM((CH,), jnp.int32),
            pltpu.VMEM((512,), jnp.int32),
            pltpu.VMEM((512,), jnp.int32),
            pltpu.VMEM((512,), jnp.int32),
            pltpu.VMEM((32,), jnp.int32),
            pltpu.VMEM((CH,), jnp.int32),
            pltpu.VMEM((CH,), jnp.int32),
            pltpu.VMEM((CH,), jnp.float32),
            pltpu.VMEM((1024,), jnp.float32),
            pltpu.VMEM((CH // 4, 128), jnp.float32),
            pltpu.VMEM((CH // 4, 128), jnp.float32),
            pltpu.VMEM((CH // 4,), jnp.int32),
            pltpu.VMEM((CH // 4,), jnp.int32),
            pltpu.VMEM_SHARED((NPAD,), jnp.int32),
            pltpu.VMEM_SHARED((NPAD,), jnp.int32),
            pltpu.VMEM_SHARED((NPAD,), jnp.int32),
            pltpu.VMEM_SHARED((NPAD,), jnp.int32),
            pltpu.VMEM_SHARED((512,), jnp.int32),
            pltpu.SemaphoreType.DMA,
            pltpu.SemaphoreType.DMA,
        ],
    )
    return f(table, rmax_p, pair0_p, pair1_p, stag)


def kernel(rel_logit, obj_logit, rel_pair_idx, boxes):
    obj_scores2d, obj_pred2d, stag, rmax2d = _tc_prep(
        rel_logit, rel_pair_idx, obj_logit)

    table = jnp.pad(obj_scores2d[:, 0], (0, 24))
    rmax_p = jnp.pad(rmax2d[:, 0], (0, NPAD - NREL))
    pair0_p = jnp.pad(rel_pair_idx[:, 0], (0, NPAD - NREL))
    pair1_p = jnp.pad(rel_pair_idx[:, 1], (0, NPAD - NREL))
    sorted_rows = _sc_sort(table, rmax_p, pair0_p, pair1_p, stag)

    rel_class_prob_sorted = sorted_rows[:NREL, :51]
    rel_max_scores_sorted = sorted_rows[:NREL, 51]
    rel_pair_sorted = lax.bitcast_convert_type(
        sorted_rows[:NREL, 52:54], jnp.int32)
    rel_labels_sorted = lax.bitcast_convert_type(
        sorted_rows[:NREL, 54], jnp.int32)
    return (boxes, obj_pred2d[:, 0], obj_scores2d[:, 0], rel_pair_sorted,
            rel_class_prob_sorted, rel_labels_sorted, rel_max_scores_sorted)
